# Initial kernel scaffold; baseline (speedup 1.0000x reference)
import jax
import jax.numpy as jnp
from jax.experimental import pallas as pl


def kernel(users, user_emb, item_emb, edge_src, edge_dst, edge_val):
    raise NotImplementedError("write your pallas kernel here")


# speedup vs baseline: 9.9303x; 9.9303x over previous
import functools

import jax
import jax.numpy as jnp
from jax import lax
from jax.experimental import pallas as pl
from jax.experimental.pallas import tpu as pltpu
from jax.experimental.pallas import tpu_sc as plsc

NU = 25000
NI = 25000
NN = NU + NI
D = 32
NINTER = 400000
NLAYERS = 3
NBATCH = 1024

NCORES = 2
NSUB = 16
EPW = NINTER // NSUB
NCHUNK = 196
EPAD = NCHUNK * 128
ACC_ROWS = 25088

_mesh = plsc.VectorSubcoreMesh(core_axis_name="c", subcore_axis_name="s")

_GDN = lax.GatherDimensionNumbers(
    offset_dims=(), collapsed_slice_dims=(0,), start_index_map=(0,))


def _splat(vec16, r_idx):
    return lax.gather(vec16, r_idx[:, None], dimension_numbers=_GDN,
                      slice_sizes=(1,),
                      mode=lax.GatherScatterMode.PROMISE_IN_BOUNDS)


def _propagate_body(tab_in, srcs, dsts, vals, zrows, tab_out,
                    src_v, dst_v, val_v, rows_v, acc, sem):
    c = lax.axis_index("c")
    s = lax.axis_index("s")

    pltpu.sync_copy(srcs.at[c, s], src_v)
    pltpu.sync_copy(dsts.at[c, s], dst_v)
    pltpu.sync_copy(vals.at[c, s], val_v)

    pltpu.sync_copy(zrows, acc.at[pl.ds(s * 1568, 1568)])
    plsc.subcore_barrier()

    r_consts = [jnp.full((16,), r, jnp.int32) for r in range(16)]

    def chunk(j, carry):
        pltpu.async_copy(tab_in.at[src_v.at[pl.ds(j * 128, 128)]],
                         rows_v, sem).wait()
        for g in range(8):
            val16 = val_v[pl.ds(j * 128 + g * 16, 16)]
            for r in range(16):
                e = g * 16 + r
                sp = _splat(val16, r_consts[r])
                lo = rows_v[e, pl.ds(0, 16)]
                hi = rows_v[e, pl.ds(16, 16)]
                rows_v[e, pl.ds(0, 16)] = lo * sp
                rows_v[e, pl.ds(16, 16)] = hi * sp
        pltpu.sync_copy(rows_v, acc.at[dst_v.at[j]], add=True)
        return carry

    lax.fori_loop(0, NCHUNK, chunk, 0)
    plsc.subcore_barrier()

    @pl.when(s < 15)
    def _():
        base = s * 1568
        pltpu.sync_copy(acc.at[pl.ds(base, 1568)],
                        tab_out.at[pl.ds(c * NU + base, 1568)])

    @pl.when(s == 15)
    def _():
        pltpu.sync_copy(acc.at[pl.ds(23520, 1480)],
                        tab_out.at[pl.ds(c * NU + 23520, 1480)])


_propagate = pl.kernel(
    _propagate_body,
    out_type=jax.ShapeDtypeStruct((NN, D), jnp.float32),
    mesh=_mesh,
    scratch_types=[
        pltpu.VMEM((EPAD,), jnp.int32),
        pltpu.VMEM((NCHUNK, 128), jnp.int32),
        pltpu.VMEM((EPAD,), jnp.float32),
        pltpu.VMEM((128, D), jnp.float32),
        pltpu.VMEM_SHARED((ACC_ROWS, D), jnp.float32),
        pltpu.SemaphoreType.DMA,
    ],
    compiler_params=pltpu.CompilerParams(use_tc_tiling_on_sc=False),
)


def _user_mean_body(users, t0, t1, t2, t3, uemb,
                    idx_v, r0, r1, r2, r3, sem):
    c = lax.axis_index("c")
    s = lax.axis_index("s")
    wid = s * NCORES + c
    base = wid * 32
    pltpu.sync_copy(users.at[pl.ds(base, 32)], idx_v)
    pltpu.async_copy(t0.at[idx_v], r0, sem).wait()
    pltpu.async_copy(t1.at[idx_v], r1, sem).wait()
    pltpu.async_copy(t2.at[idx_v], r2, sem).wait()
    pltpu.async_copy(t3.at[idx_v], r3, sem).wait()
    for i in range(32):
        for h in range(2):
            sl = pl.ds(h * 16, 16)
            m = (r0[i, sl] + r1[i, sl] + r2[i, sl] + r3[i, sl]) * 0.25
            r0[i, sl] = m
    pltpu.sync_copy(r0, uemb.at[pl.ds(base, 32)])


_user_mean = pl.kernel(
    _user_mean_body,
    out_type=jax.ShapeDtypeStruct((NBATCH, D), jnp.float32),
    mesh=_mesh,
    scratch_types=[
        pltpu.VMEM((32,), jnp.int32),
        pltpu.VMEM((32, D), jnp.float32),
        pltpu.VMEM((32, D), jnp.float32),
        pltpu.VMEM((32, D), jnp.float32),
        pltpu.VMEM((32, D), jnp.float32),
        pltpu.SemaphoreType.DMA,
    ],
    compiler_params=pltpu.CompilerParams(use_tc_tiling_on_sc=False),
)

BN = 512
NIPAD = 25088


def _rating_body(u_ref, i0, i1, i2, i3, out_ref):
    u = u_ref[...]
    m = (i0[...] + i1[...] + i2[...] + i3[...]) * 0.25
    x = lax.dot_general(u, m, (((1,), (1,)), ((), ())),
                        preferred_element_type=jnp.float32)
    out_ref[...] = 1.0 / (1.0 + jnp.exp(-x))


@functools.partial(jax.jit, static_argnames=())
def _rating(uemb, it0, it1, it2, it3):
    return pl.pallas_call(
        _rating_body,
        grid=(NIPAD // BN,),
        in_specs=[
            pl.BlockSpec((NBATCH, D), lambda j: (0, 0)),
            pl.BlockSpec((BN, D), lambda j: (j, 0)),
            pl.BlockSpec((BN, D), lambda j: (j, 0)),
            pl.BlockSpec((BN, D), lambda j: (j, 0)),
            pl.BlockSpec((BN, D), lambda j: (j, 0)),
        ],
        out_specs=pl.BlockSpec((NBATCH, BN), lambda j: (0, j)),
        out_shape=jax.ShapeDtypeStruct((NBATCH, NI), jnp.float32),
    )(uemb, it0, it1, it2, it3)


def kernel(users, user_emb, item_emb, edge_src, edge_dst, edge_val):
    users_i = users.astype(jnp.int32)
    src = edge_src.astype(jnp.int32)
    dst = edge_dst.astype(jnp.int32)
    val = edge_val.astype(jnp.float32)

    def group(a):
        return jnp.stack([a[NINTER:], a[:NINTER]]).reshape(NCORES, NSUB, EPW)

    pad_src = jnp.broadcast_to(
        (jnp.arange(88, dtype=jnp.int32) * 571) % NN, (NCORES, NSUB, 88))
    pad_dst = jnp.broadcast_to(
        (jnp.arange(88, dtype=jnp.int32) * 37) % NU, (NCORES, NSUB, 88))
    pad_val = jnp.zeros((NCORES, NSUB, 88), jnp.float32)

    srcs = jnp.concatenate([group(src), pad_src], axis=-1)
    dst_local = group(dst) - jnp.array([0, NU], jnp.int32)[:, None, None]
    dsts = jnp.concatenate([dst_local, pad_dst], axis=-1)
    dsts = dsts.reshape(NCORES, NSUB, NCHUNK, 128)
    vals = jnp.concatenate([group(val), pad_val], axis=-1)

    zrows = jnp.zeros((1568, D), jnp.float32)

    tabs = [jnp.concatenate([user_emb, item_emb], axis=0)]
    for _ in range(NLAYERS):
        tabs.append(_propagate(tabs[-1], srcs, dsts, vals, zrows))

    uemb = _user_mean(users_i, *tabs)
    its = [jnp.pad(t[NU:], ((0, NIPAD - NI), (0, 0))) for t in tabs]
    return _rating(uemb, *its)


# speedup vs baseline: 10.3992x; 1.0472x over previous
import functools

import jax
import jax.numpy as jnp
from jax import lax
from jax.experimental import pallas as pl
from jax.experimental.pallas import tpu as pltpu
from jax.experimental.pallas import tpu_sc as plsc

NU = 25000
NI = 25000
NN = NU + NI
D = 32
NINTER = 400000
NLAYERS = 3
NBATCH = 1024

NCORES = 2
NSUB = 16
EPW = NINTER // NSUB
NCHUNK = 196
EPAD = NCHUNK * 128
ACC_ROWS = 25088

_mesh = plsc.VectorSubcoreMesh(core_axis_name="c", subcore_axis_name="s")

_GDN = lax.GatherDimensionNumbers(
    offset_dims=(), collapsed_slice_dims=(0,), start_index_map=(0,))


def _splat(vec16, r_idx):
    return lax.gather(vec16, r_idx[:, None], dimension_numbers=_GDN,
                      slice_sizes=(1,),
                      mode=lax.GatherScatterMode.PROMISE_IN_BOUNDS)


def _propagate_body(tab_in, srcs, dsts, vals, zrows, tab_out,
                    src_v, dst_v, val_d, rows_v, acc, sem, vsem):
    c = lax.axis_index("c")
    s = lax.axis_index("s")

    pltpu.sync_copy(srcs.at[c, s], src_v)
    pltpu.sync_copy(dsts.at[c, s], dst_v)

    pltpu.sync_copy(zrows, acc.at[pl.ds(s * 1568, 1568)])
    plsc.subcore_barrier()

    r_consts = [jnp.full((16,), r, jnp.int32) for r in range(16)]

    def start(cidx, buf, b):
        pltpu.async_copy(tab_in.at[src_v.at[pl.ds(cidx * 128, 128)]],
                         buf, sem)
        pltpu.async_copy(vals.at[c, s, pl.ds(cidx * 128, 128)],
                         val_d.at[b], vsem)

    def drain(buf, b):
        pltpu.make_async_copy(tab_in.at[pl.ds(0, 128)], buf, sem).wait()
        pltpu.make_async_copy(vals.at[0, 0, pl.ds(0, 128)],
                              val_d.at[b], vsem).wait()

    def scale_scatter(cidx, buf, b):
        for g in range(8):
            val16 = val_d[b, pl.ds(g * 16, 16)]
            for r in range(16):
                e = g * 16 + r
                sp = _splat(val16, r_consts[r])
                lo = buf[e, pl.ds(0, 16)]
                hi = buf[e, pl.ds(16, 16)]
                buf[e, pl.ds(0, 16)] = lo * sp
                buf[e, pl.ds(16, 16)] = hi * sp
        pltpu.sync_copy(buf, acc.at[dst_v.at[cidx]], add=True)

    start(0, rows_v.at[0], 0)
    start(1, rows_v.at[1], 1)

    def pair(j, carry):
        drain(rows_v.at[0], 0)
        scale_scatter(2 * j, rows_v.at[0], 0)

        @pl.when(j < NCHUNK // 2 - 1)
        def _():
            start(2 * j + 2, rows_v.at[0], 0)

        drain(rows_v.at[1], 1)
        scale_scatter(2 * j + 1, rows_v.at[1], 1)

        @pl.when(j < NCHUNK // 2 - 1)
        def _():
            start(2 * j + 3, rows_v.at[1], 1)

        return carry

    lax.fori_loop(0, NCHUNK // 2, pair, 0)
    plsc.subcore_barrier()

    @pl.when(s < 15)
    def _():
        base = s * 1568
        pltpu.sync_copy(acc.at[pl.ds(base, 1568)],
                        tab_out.at[pl.ds(c * NU + base, 1568)])

    @pl.when(s == 15)
    def _():
        pltpu.sync_copy(acc.at[pl.ds(23520, 1480)],
                        tab_out.at[pl.ds(c * NU + 23520, 1480)])


_propagate = pl.kernel(
    _propagate_body,
    out_type=jax.ShapeDtypeStruct((NN, D), jnp.float32),
    mesh=_mesh,
    scratch_types=[
        pltpu.VMEM((EPAD,), jnp.int32),
        pltpu.VMEM((NCHUNK, 128), jnp.int32),
        pltpu.VMEM((2, 128), jnp.float32),
        pltpu.VMEM((2, 128, D), jnp.float32),
        pltpu.VMEM_SHARED((ACC_ROWS, D), jnp.float32),
        pltpu.SemaphoreType.DMA,
        pltpu.SemaphoreType.DMA,
    ],
    compiler_params=pltpu.CompilerParams(use_tc_tiling_on_sc=False),
)


def _user_mean_body(users, t0, t1, t2, t3, uemb,
                    idx_v, r0, r1, r2, r3, sem):
    c = lax.axis_index("c")
    s = lax.axis_index("s")
    wid = s * NCORES + c
    base = wid * 32
    pltpu.sync_copy(users.at[pl.ds(base, 32)], idx_v)
    pltpu.async_copy(t0.at[idx_v], r0, sem).wait()
    pltpu.async_copy(t1.at[idx_v], r1, sem).wait()
    pltpu.async_copy(t2.at[idx_v], r2, sem).wait()
    pltpu.async_copy(t3.at[idx_v], r3, sem).wait()
    for i in range(32):
        for h in range(2):
            sl = pl.ds(h * 16, 16)
            m = (r0[i, sl] + r1[i, sl] + r2[i, sl] + r3[i, sl]) * 0.25
            r0[i, sl] = m
    pltpu.sync_copy(r0, uemb.at[pl.ds(base, 32)])


_user_mean = pl.kernel(
    _user_mean_body,
    out_type=jax.ShapeDtypeStruct((NBATCH, D), jnp.float32),
    mesh=_mesh,
    scratch_types=[
        pltpu.VMEM((32,), jnp.int32),
        pltpu.VMEM((32, D), jnp.float32),
        pltpu.VMEM((32, D), jnp.float32),
        pltpu.VMEM((32, D), jnp.float32),
        pltpu.VMEM((32, D), jnp.float32),
        pltpu.SemaphoreType.DMA,
    ],
    compiler_params=pltpu.CompilerParams(use_tc_tiling_on_sc=False),
)

BN = 512
NIPAD = 25088


def _rating_body(u_ref, i0, i1, i2, i3, out_ref):
    u = u_ref[...]
    m = (i0[...] + i1[...] + i2[...] + i3[...]) * 0.25
    x = lax.dot_general(u, m, (((1,), (1,)), ((), ())),
                        preferred_element_type=jnp.float32)
    out_ref[...] = 1.0 / (1.0 + jnp.exp(-x))


@functools.partial(jax.jit, static_argnames=())
def _rating(uemb, it0, it1, it2, it3):
    return pl.pallas_call(
        _rating_body,
        grid=(NIPAD // BN,),
        in_specs=[
            pl.BlockSpec((NBATCH, D), lambda j: (0, 0)),
            pl.BlockSpec((BN, D), lambda j: (j, 0)),
            pl.BlockSpec((BN, D), lambda j: (j, 0)),
            pl.BlockSpec((BN, D), lambda j: (j, 0)),
            pl.BlockSpec((BN, D), lambda j: (j, 0)),
        ],
        out_specs=pl.BlockSpec((NBATCH, BN), lambda j: (0, j)),
        out_shape=jax.ShapeDtypeStruct((NBATCH, NI), jnp.float32),
    )(uemb, it0, it1, it2, it3)


def kernel(users, user_emb, item_emb, edge_src, edge_dst, edge_val):
    users_i = users.astype(jnp.int32)
    src = edge_src.astype(jnp.int32)
    dst = edge_dst.astype(jnp.int32)
    val = edge_val.astype(jnp.float32)

    def group(a):
        return jnp.stack([a[NINTER:], a[:NINTER]]).reshape(NCORES, NSUB, EPW)

    pad_src = jnp.broadcast_to(
        (jnp.arange(88, dtype=jnp.int32) * 571) % NN, (NCORES, NSUB, 88))
    pad_dst = jnp.broadcast_to(
        (jnp.arange(88, dtype=jnp.int32) * 37) % NU, (NCORES, NSUB, 88))
    pad_val = jnp.zeros((NCORES, NSUB, 88), jnp.float32)

    srcs = jnp.concatenate([group(src), pad_src], axis=-1)
    dst_local = group(dst) - jnp.array([0, NU], jnp.int32)[:, None, None]
    dsts = jnp.concatenate([dst_local, pad_dst], axis=-1)
    dsts = dsts.reshape(NCORES, NSUB, NCHUNK, 128)
    vals = jnp.concatenate([group(val), pad_val], axis=-1)

    zrows = jnp.zeros((1568, D), jnp.float32)

    tab0 = jnp.concatenate([user_emb, item_emb], axis=0)

    def step(tab, _):
        nt = _propagate(tab, srcs, dsts, vals, zrows)
        return nt, nt

    _, ys = lax.scan(step, tab0, None, length=NLAYERS)
    tabs = [tab0, ys[0], ys[1], ys[2]]

    uemb = _user_mean(users_i, *tabs)
    its = [jnp.pad(t[NU:], ((0, NIPAD - NI), (0, 0))) for t in tabs]
    return _rating(uemb, *its)


# speedup vs baseline: 11.7246x; 1.1275x over previous
import functools

import jax
import jax.numpy as jnp
from jax import lax
from jax.experimental import pallas as pl
from jax.experimental.pallas import tpu as pltpu
from jax.experimental.pallas import tpu_sc as plsc

NU = 25000
NI = 25000
NN = NU + NI
D = 32
NINTER = 400000
NLAYERS = 3
NBATCH = 1024

NCORES = 2
NSUB = 16
EPW = NINTER // NSUB
NCHUNK = 196
EPAD = NCHUNK * 128
ACC_ROWS = 25088

_mesh = plsc.VectorSubcoreMesh(core_axis_name="c", subcore_axis_name="s")

_GDN = lax.GatherDimensionNumbers(
    offset_dims=(), collapsed_slice_dims=(0,), start_index_map=(0,))


def _splat(vec16, r_idx):
    return lax.gather(vec16, r_idx[:, None], dimension_numbers=_GDN,
                      slice_sizes=(1,),
                      mode=lax.GatherScatterMode.PROMISE_IN_BOUNDS)


def _propagate_body(tab_in, srcs, dsts, vals, zrows, tab_out,
                    src_v, dst_v, val_d, rows_v, acc, sem, vsem,
                    ssem0, ssem1, ssem2, ssem3):
    c = lax.axis_index("c")
    s = lax.axis_index("s")

    pltpu.sync_copy(srcs.at[c, s], src_v)
    pltpu.sync_copy(dsts.at[c, s], dst_v)

    pltpu.sync_copy(zrows, acc.at[pl.ds(s * 1568, 1568)])
    plsc.subcore_barrier()

    r_consts = [jnp.full((16,), r, jnp.int32) for r in range(16)]

    def start(cidx, buf, b):
        pltpu.async_copy(tab_in.at[src_v.at[pl.ds(cidx * 128, 128)]],
                         buf, sem)
        pltpu.async_copy(vals.at[c, s, pl.ds(cidx * 128, 128)],
                         val_d.at[b], vsem)

    def drain(buf, b):
        pltpu.make_async_copy(tab_in.at[pl.ds(0, 128)], buf, sem).wait()
        pltpu.make_async_copy(vals.at[0, 0, pl.ds(0, 128)],
                              val_d.at[b], vsem).wait()

    ssems = [ssem0, ssem1, ssem2, ssem3]

    def scale(cidx, buf, b):
        for g in range(8):
            val16 = val_d[b, pl.ds(g * 16, 16)]
            for r in range(16):
                e = g * 16 + r
                sp = _splat(val16, r_consts[r])
                lo = buf[e, pl.ds(0, 16)]
                hi = buf[e, pl.ds(16, 16)]
                buf[e, pl.ds(0, 16)] = lo * sp
                buf[e, pl.ds(16, 16)] = hi * sp

    def scat_wait(b):
        pltpu.make_async_copy(tab_in.at[pl.ds(0, 128)],
                              rows_v.at[b], ssems[b]).wait()

    for b in range(4):
        start(b, rows_v.at[b], b)

    def group(j, carry):
        for b in range(4):
            cidx = 4 * j + b
            drain(rows_v.at[b], b)
            scale(cidx, rows_v.at[b], b)
            pltpu.async_copy(rows_v.at[b], acc.at[dst_v.at[cidx]],
                             ssems[b], add=True)

            @pl.when(cidx < NCHUNK - 4)
            def _():
                scat_wait(b)
                start(cidx + 4, rows_v.at[b], b)

        return carry

    lax.fori_loop(0, NCHUNK // 4, group, 0)
    for b in range(4):
        scat_wait(b)
    plsc.subcore_barrier()

    @pl.when(s < 15)
    def _():
        base = s * 1568
        pltpu.sync_copy(acc.at[pl.ds(base, 1568)],
                        tab_out.at[pl.ds(c * NU + base, 1568)])

    @pl.when(s == 15)
    def _():
        pltpu.sync_copy(acc.at[pl.ds(23520, 1480)],
                        tab_out.at[pl.ds(c * NU + 23520, 1480)])


_propagate = pl.kernel(
    _propagate_body,
    out_type=jax.ShapeDtypeStruct((NN, D), jnp.float32),
    mesh=_mesh,
    scratch_types=[
        pltpu.VMEM((EPAD,), jnp.int32),
        pltpu.VMEM((NCHUNK, 128), jnp.int32),
        pltpu.VMEM((4, 128), jnp.float32),
        pltpu.VMEM((4, 128, D), jnp.float32),
        pltpu.VMEM_SHARED((ACC_ROWS, D), jnp.float32),
        pltpu.SemaphoreType.DMA,
        pltpu.SemaphoreType.DMA,
        pltpu.SemaphoreType.DMA,
        pltpu.SemaphoreType.DMA,
        pltpu.SemaphoreType.DMA,
        pltpu.SemaphoreType.DMA,
    ],
    compiler_params=pltpu.CompilerParams(use_tc_tiling_on_sc=False),
)


def _user_mean_body(users, t0, t1, t2, t3, uemb,
                    idx_v, r0, r1, r2, r3, sem):
    c = lax.axis_index("c")
    s = lax.axis_index("s")
    wid = s * NCORES + c
    base = wid * 32
    pltpu.sync_copy(users.at[pl.ds(base, 32)], idx_v)
    pltpu.async_copy(t0.at[idx_v], r0, sem).wait()
    pltpu.async_copy(t1.at[idx_v], r1, sem).wait()
    pltpu.async_copy(t2.at[idx_v], r2, sem).wait()
    pltpu.async_copy(t3.at[idx_v], r3, sem).wait()
    for i in range(32):
        for h in range(2):
            sl = pl.ds(h * 16, 16)
            m = (r0[i, sl] + r1[i, sl] + r2[i, sl] + r3[i, sl]) * 0.25
            r0[i, sl] = m
    pltpu.sync_copy(r0, uemb.at[pl.ds(base, 32)])


_user_mean = pl.kernel(
    _user_mean_body,
    out_type=jax.ShapeDtypeStruct((NBATCH, D), jnp.float32),
    mesh=_mesh,
    scratch_types=[
        pltpu.VMEM((32,), jnp.int32),
        pltpu.VMEM((32, D), jnp.float32),
        pltpu.VMEM((32, D), jnp.float32),
        pltpu.VMEM((32, D), jnp.float32),
        pltpu.VMEM((32, D), jnp.float32),
        pltpu.SemaphoreType.DMA,
    ],
    compiler_params=pltpu.CompilerParams(use_tc_tiling_on_sc=False),
)

BN = 512
NIPAD = 25088


def _rating_body(u_ref, i0, i1, i2, i3, out_ref):
    u = u_ref[...]
    m = (i0[...] + i1[...] + i2[...] + i3[...]) * 0.25
    x = lax.dot_general(u, m, (((1,), (1,)), ((), ())),
                        preferred_element_type=jnp.float32)
    out_ref[...] = 1.0 / (1.0 + jnp.exp(-x))


@functools.partial(jax.jit, static_argnames=())
def _rating(uemb, it0, it1, it2, it3):
    return pl.pallas_call(
        _rating_body,
        grid=(NIPAD // BN,),
        in_specs=[
            pl.BlockSpec((NBATCH, D), lambda j: (0, 0)),
            pl.BlockSpec((BN, D), lambda j: (j, 0)),
            pl.BlockSpec((BN, D), lambda j: (j, 0)),
            pl.BlockSpec((BN, D), lambda j: (j, 0)),
            pl.BlockSpec((BN, D), lambda j: (j, 0)),
        ],
        out_specs=pl.BlockSpec((NBATCH, BN), lambda j: (0, j)),
        out_shape=jax.ShapeDtypeStruct((NBATCH, NI), jnp.float32),
    )(uemb, it0, it1, it2, it3)


def kernel(users, user_emb, item_emb, edge_src, edge_dst, edge_val):
    users_i = users.astype(jnp.int32)
    src = edge_src.astype(jnp.int32)
    dst = edge_dst.astype(jnp.int32)
    val = edge_val.astype(jnp.float32)

    def group(a):
        return jnp.stack([a[NINTER:], a[:NINTER]]).reshape(NCORES, NSUB, EPW)

    pad_src = jnp.broadcast_to(
        (jnp.arange(88, dtype=jnp.int32) * 571) % NN, (NCORES, NSUB, 88))
    pad_dst = jnp.broadcast_to(
        (jnp.arange(88, dtype=jnp.int32) * 37) % NU, (NCORES, NSUB, 88))
    pad_val = jnp.zeros((NCORES, NSUB, 88), jnp.float32)

    srcs = jnp.concatenate([group(src), pad_src], axis=-1)
    dst_local = group(dst) - jnp.array([0, NU], jnp.int32)[:, None, None]
    dsts = jnp.concatenate([dst_local, pad_dst], axis=-1)
    dsts = dsts.reshape(NCORES, NSUB, NCHUNK, 128)
    vals = jnp.concatenate([group(val), pad_val], axis=-1)

    zrows = jnp.zeros((1568, D), jnp.float32)

    tab0 = jnp.concatenate([user_emb, item_emb], axis=0)

    def step(tab, _):
        nt = _propagate(tab, srcs, dsts, vals, zrows)
        return nt, nt

    _, ys = lax.scan(step, tab0, None, length=NLAYERS)
    tabs = [tab0, ys[0], ys[1], ys[2]]

    uemb = _user_mean(users_i, *tabs)
    its = [jnp.pad(t[NU:], ((0, NIPAD - NI), (0, 0))) for t in tabs]
    return _rating(uemb, *its)
